# baseline (speedup 1.0000x reference)
import functools

import jax
import jax.numpy as jnp
from jax import lax
from jax.experimental import pallas as pl
from jax.experimental.pallas import tpu as pltpu
from jax.experimental.pallas import tpu_sc as plsc

DIM = 128
CHUNK = 80
NBUF = 3
LAG = 2
HBM_CHUNKS = 3


@functools.cache
def _build(n_rows, n_table_rows, dim):
    info = plsc.get_sparse_core_info()
    nc, ns = info.num_cores, info.num_subcores
    nw = nc * ns
    n_chunks = n_rows // (nw * CHUNK)
    assert n_chunks * nw * CHUNK == n_rows
    assert HBM_CHUNKS % NBUF == 0 and LAG <= NBUF <= HBM_CHUNKS
    n_rest = n_chunks - HBM_CHUNKS
    n_main = (n_rest // NBUF) * NBUF

    pre_rows = -(-n_table_rows // (8 * ns)) * 8
    pre_last = n_table_rows - pre_rows
    assert pre_rows % 8 == 0 and pre_last % 8 == 0 and pre_last >= 0

    mesh = plsc.VectorSubcoreMesh(core_axis_name="c", subcore_axis_name="s")

    @functools.partial(
        pl.kernel,
        mesh=mesh,
        out_type=jax.ShapeDtypeStruct((nw, n_chunks, CHUNK, dim), jnp.float32),
        scratch_types=[
            pltpu.VMEM((n_chunks, CHUNK), jnp.int32),
            pltpu.VMEM((NBUF, CHUNK, dim), jnp.float32),
            pltpu.VMEM_SHARED((n_table_rows, dim), jnp.float32),
            pltpu.SemaphoreType.DMA,
        ]
        + [pltpu.SemaphoreType.DMA] * (2 * NBUF),
    )
    def gather_kernel(idx_hbm, table_hbm, out_hbm, idx_v, rows_v, table_sh, psem, *sems):
        gsem, wsem = sems[:NBUF], sems[NBUF:]
        sid = lax.axis_index("s")
        wid = sid * nc + lax.axis_index("c")

        pre_start = pl.multiple_of(jnp.minimum(sid * pre_rows, pre_last), 8)
        pltpu.async_copy(
            table_hbm.at[pl.ds(pre_start, pre_rows)],
            table_sh.at[pl.ds(pre_start, pre_rows)],
            psem,
        )

        pltpu.sync_copy(idx_hbm.at[wid], idx_v)

        def gfire_hbm(c, b):
            pltpu.async_copy(table_hbm.at[idx_v.at[c]], rows_v.at[b], gsem[b])

        def gfire_sp(c, b):
            pltpu.async_copy(table_sh.at[idx_v.at[c]], rows_v.at[b], gsem[b])

        def gwait(b):
            pltpu.make_async_copy(
                table_hbm.at[idx_v.at[0]], rows_v.at[b], gsem[b]
            ).wait()

        def wfire(c, b):
            pltpu.async_copy(rows_v.at[b], out_hbm.at[wid, c], wsem[b])

        def wwait(b):
            pltpu.make_async_copy(rows_v.at[b], out_hbm.at[wid, 0], wsem[b]).wait()

        for b in range(LAG):
            gfire_hbm(b, b)

        def static_step(c, gfire_fn):
            b = c % NBUF
            gwait(b)
            wfire(c, b)
            if c + LAG < n_chunks:
                bn = (b + LAG) % NBUF
                if c + LAG >= NBUF:
                    wwait(bn)
                gfire_fn(c + LAG, bn)

        for c in range(HBM_CHUNKS - LAG):
            static_step(c, gfire_hbm)

        pltpu.make_async_copy(
            table_hbm.at[pl.ds(0, pre_rows)],
            table_sh.at[pl.ds(0, pre_rows)],
            psem,
        ).wait()
        plsc.subcore_barrier()

        for c in range(HBM_CHUNKS - LAG, HBM_CHUNKS):
            static_step(c, gfire_sp)

        def body(j, carry):
            for b in range(NBUF):
                c = HBM_CHUNKS + j * NBUF + b
                gwait(b)
                wfire(c, b)
                bn = (b + LAG) % NBUF

                @pl.when(c + LAG < n_chunks)
                def _():
                    wwait(bn)
                    gfire_sp(c + LAG, bn)

            return carry

        lax.fori_loop(0, n_main // NBUF, body, 0, unroll=False)
        for c in range(HBM_CHUNKS + n_main, n_chunks):
            static_step(c, gfire_sp)
        for b in range(NBUF):
            wwait(b)

    return gather_kernel, nw, n_chunks


def kernel(x, embedding):
    b, h = x.shape
    n_table_rows, dim = embedding.shape
    n_rows = b * h
    gather_kernel, nw, n_chunks = _build(n_rows, n_table_rows, dim)
    idx = x.reshape(nw, n_chunks, CHUNK)
    out = gather_kernel(idx, embedding)
    return out.reshape(b, h, dim)


# speedup vs baseline: 1.0339x; 1.0339x over previous
import functools

import jax
import jax.numpy as jnp
from jax import lax
from jax.experimental import pallas as pl
from jax.experimental.pallas import tpu as pltpu
from jax.experimental.pallas import tpu_sc as plsc

DIM = 128
CHUNK = 128
NBUF = 2
LAG = 2
HBM_CHUNKS = 4


@functools.cache
def _build(n_rows, n_table_rows, dim):
    info = plsc.get_sparse_core_info()
    nc, ns = info.num_cores, info.num_subcores
    nw = nc * ns
    n_chunks = n_rows // (nw * CHUNK)
    assert n_chunks * nw * CHUNK == n_rows
    assert HBM_CHUNKS % NBUF == 0 and LAG <= NBUF <= HBM_CHUNKS
    n_rest = n_chunks - HBM_CHUNKS
    n_main = (n_rest // NBUF) * NBUF

    pre_rows = -(-n_table_rows // (8 * ns)) * 8
    pre_last = n_table_rows - pre_rows
    assert pre_rows % 8 == 0 and pre_last % 8 == 0 and pre_last >= 0

    mesh = plsc.VectorSubcoreMesh(core_axis_name="c", subcore_axis_name="s")

    @functools.partial(
        pl.kernel,
        mesh=mesh,
        out_type=jax.ShapeDtypeStruct((nw, n_chunks, CHUNK, dim), jnp.float32),
        scratch_types=[
            pltpu.VMEM((n_chunks, CHUNK), jnp.int32),
            pltpu.VMEM((NBUF, CHUNK, dim), jnp.float32),
            pltpu.VMEM_SHARED((n_table_rows, dim), jnp.float32),
            pltpu.SemaphoreType.DMA,
        ]
        + [pltpu.SemaphoreType.DMA] * (2 * NBUF),
    )
    def gather_kernel(idx_hbm, table_hbm, out_hbm, idx_v, rows_v, table_sh, psem, *sems):
        gsem, wsem = sems[:NBUF], sems[NBUF:]
        sid = lax.axis_index("s")
        wid = sid * nc + lax.axis_index("c")

        pre_start = pl.multiple_of(jnp.minimum(sid * pre_rows, pre_last), 8)
        pltpu.async_copy(
            table_hbm.at[pl.ds(pre_start, pre_rows)],
            table_sh.at[pl.ds(pre_start, pre_rows)],
            psem,
        )

        pltpu.sync_copy(idx_hbm.at[wid], idx_v)

        def gfire_hbm(c, b):
            pltpu.async_copy(table_hbm.at[idx_v.at[c]], rows_v.at[b], gsem[b])

        def gfire_sp(c, b):
            pltpu.async_copy(table_sh.at[idx_v.at[c]], rows_v.at[b], gsem[b])

        def gwait(b):
            pltpu.make_async_copy(
                table_hbm.at[idx_v.at[0]], rows_v.at[b], gsem[b]
            ).wait()

        def wfire(c, b):
            pltpu.async_copy(rows_v.at[b], out_hbm.at[wid, c], wsem[b])

        def wwait(b):
            pltpu.make_async_copy(rows_v.at[b], out_hbm.at[wid, 0], wsem[b]).wait()

        for b in range(LAG):
            gfire_hbm(b, b)

        def static_step(c, gfire_fn):
            b = c % NBUF
            gwait(b)
            wfire(c, b)
            if c + LAG < n_chunks:
                bn = (b + LAG) % NBUF
                if c + LAG >= NBUF:
                    wwait(bn)
                gfire_fn(c + LAG, bn)

        for c in range(HBM_CHUNKS - LAG):
            static_step(c, gfire_hbm)

        pltpu.make_async_copy(
            table_hbm.at[pl.ds(0, pre_rows)],
            table_sh.at[pl.ds(0, pre_rows)],
            psem,
        ).wait()
        plsc.subcore_barrier()

        for c in range(HBM_CHUNKS - LAG, HBM_CHUNKS):
            static_step(c, gfire_sp)

        def body(j, carry):
            for b in range(NBUF):
                c = HBM_CHUNKS + j * NBUF + b
                gwait(b)
                wfire(c, b)
                bn = (b + LAG) % NBUF

                @pl.when(c + LAG < n_chunks)
                def _():
                    wwait(bn)
                    gfire_sp(c + LAG, bn)

            return carry

        lax.fori_loop(0, n_main // NBUF, body, 0, unroll=False)
        for c in range(HBM_CHUNKS + n_main, n_chunks):
            static_step(c, gfire_sp)
        for b in range(NBUF):
            wwait(b)

    return gather_kernel, nw, n_chunks


def kernel(x, embedding):
    b, h = x.shape
    n_table_rows, dim = embedding.shape
    n_rows = b * h
    gather_kernel, nw, n_chunks = _build(n_rows, n_table_rows, dim)
    idx = x.reshape(nw, n_chunks, CHUNK)
    out = gather_kernel(idx, embedding)
    return out.reshape(b, h, dim)


# speedup vs baseline: 1.0384x; 1.0043x over previous
import functools

import jax
import jax.numpy as jnp
from jax import lax
from jax.experimental import pallas as pl
from jax.experimental.pallas import tpu as pltpu
from jax.experimental.pallas import tpu_sc as plsc

DIM = 128
CHUNK = 128
NBUF = 2
LAG = 2
HBM_CHUNKS = 2


@functools.cache
def _build(n_rows, n_table_rows, dim):
    info = plsc.get_sparse_core_info()
    nc, ns = info.num_cores, info.num_subcores
    nw = nc * ns
    n_chunks = n_rows // (nw * CHUNK)
    assert n_chunks * nw * CHUNK == n_rows
    assert HBM_CHUNKS % NBUF == 0 and LAG <= NBUF <= HBM_CHUNKS
    n_rest = n_chunks - HBM_CHUNKS
    n_main = (n_rest // NBUF) * NBUF

    pre_rows = -(-n_table_rows // (8 * ns)) * 8
    pre_last = n_table_rows - pre_rows
    assert pre_rows % 8 == 0 and pre_last % 8 == 0 and pre_last >= 0

    mesh = plsc.VectorSubcoreMesh(core_axis_name="c", subcore_axis_name="s")

    @functools.partial(
        pl.kernel,
        mesh=mesh,
        out_type=jax.ShapeDtypeStruct((nw, n_chunks, CHUNK, dim), jnp.float32),
        scratch_types=[
            pltpu.VMEM((n_chunks, CHUNK), jnp.int32),
            pltpu.VMEM((NBUF, CHUNK, dim), jnp.float32),
            pltpu.VMEM_SHARED((n_table_rows, dim), jnp.float32),
            pltpu.SemaphoreType.DMA,
        ]
        + [pltpu.SemaphoreType.DMA] * (2 * NBUF),
    )
    def gather_kernel(idx_hbm, table_hbm, out_hbm, idx_v, rows_v, table_sh, psem, *sems):
        gsem, wsem = sems[:NBUF], sems[NBUF:]
        sid = lax.axis_index("s")
        wid = sid * nc + lax.axis_index("c")

        pre_start = pl.multiple_of(jnp.minimum(sid * pre_rows, pre_last), 8)
        pltpu.async_copy(
            table_hbm.at[pl.ds(pre_start, pre_rows)],
            table_sh.at[pl.ds(pre_start, pre_rows)],
            psem,
        )

        pltpu.sync_copy(idx_hbm.at[wid], idx_v)

        def gfire_hbm(c, b):
            pltpu.async_copy(table_hbm.at[idx_v.at[c]], rows_v.at[b], gsem[b])

        def gfire_sp(c, b):
            pltpu.async_copy(table_sh.at[idx_v.at[c]], rows_v.at[b], gsem[b])

        def gwait(b):
            pltpu.make_async_copy(
                table_hbm.at[idx_v.at[0]], rows_v.at[b], gsem[b]
            ).wait()

        def wfire(c, b):
            pltpu.async_copy(rows_v.at[b], out_hbm.at[wid, c], wsem[b])

        def wwait(b):
            pltpu.make_async_copy(rows_v.at[b], out_hbm.at[wid, 0], wsem[b]).wait()

        for b in range(LAG):
            gfire_hbm(b, b)

        def static_step(c, gfire_fn):
            b = c % NBUF
            gwait(b)
            wfire(c, b)
            if c + LAG < n_chunks:
                bn = (b + LAG) % NBUF
                if c + LAG >= NBUF:
                    wwait(bn)
                gfire_fn(c + LAG, bn)

        for c in range(HBM_CHUNKS - LAG):
            static_step(c, gfire_hbm)

        pltpu.make_async_copy(
            table_hbm.at[pl.ds(0, pre_rows)],
            table_sh.at[pl.ds(0, pre_rows)],
            psem,
        ).wait()
        plsc.subcore_barrier()

        for c in range(HBM_CHUNKS - LAG, HBM_CHUNKS):
            static_step(c, gfire_sp)

        def body(j, carry):
            for b in range(NBUF):
                c = HBM_CHUNKS + j * NBUF + b
                gwait(b)
                wfire(c, b)
                bn = (b + LAG) % NBUF

                @pl.when(c + LAG < n_chunks)
                def _():
                    wwait(bn)
                    gfire_sp(c + LAG, bn)

            return carry

        lax.fori_loop(0, n_main // NBUF, body, 0, unroll=2)
        for c in range(HBM_CHUNKS + n_main, n_chunks):
            static_step(c, gfire_sp)
        for b in range(NBUF):
            wwait(b)

    return gather_kernel, nw, n_chunks


def kernel(x, embedding):
    b, h = x.shape
    n_table_rows, dim = embedding.shape
    n_rows = b * h
    gather_kernel, nw, n_chunks = _build(n_rows, n_table_rows, dim)
    idx = x.reshape(nw, n_chunks, CHUNK)
    out = gather_kernel(idx, embedding)
    return out.reshape(b, h, dim)
